# Initial kernel scaffold; baseline (speedup 1.0000x reference)
import jax
import jax.numpy as jnp
from jax.experimental import pallas as pl


def kernel(x, edge_index, batch, W_l, b_l, W_r, W2, b2):
    raise NotImplementedError("write your pallas kernel here")


# speedup vs baseline: 6.1224x; 6.1224x over previous
import jax
import jax.numpy as jnp
from jax import lax
from jax.experimental import pallas as pl
from jax.experimental.pallas import tpu as pltpu
from jax.experimental.pallas import tpu_sc as plsc

N = 10000
E = 320000
F = 128
HF = 64
NG = 64
NCLS = 8

NC, NS = 2, 16
CH = 128
RPW = 157
EPAD = NS * RPW * CH - E
TBL = 10240
RPT = TBL // NS
DST_PAD = N + 100

NBLK = 79
NPAD = NBLK * 128


def _sc_body(x2_hbm, src_hbm, dst_hbm, sums_out, cnt_out,
             sidx_v, didx_v, rows_v, ones_v, zero16_v, sums_sh, cnt_sh, sem):
    c = lax.axis_index("c")
    s = lax.axis_index("s")

    def fill(r, carry):
        for cc in range(HF // 16):
            rows_v[r, pl.ds(cc * 16, 16)] = jnp.zeros((16,), jnp.float32)
        ones_v[r, :] = jnp.ones((16,), jnp.float32)
        zero16_v[r, :] = jnp.zeros((16,), jnp.float32)
        return carry

    lax.fori_loop(0, CH, fill, 0)

    for k in range(RPT // CH):
        pltpu.sync_copy(rows_v, sums_sh.at[pl.ds(s * RPT + k * CH, CH)])

    @pl.when(c == 0)
    def _zero_cnt():
        for k in range(RPT // CH):
            pltpu.sync_copy(zero16_v, cnt_sh.at[pl.ds(s * RPT + k * CH, CH)])

    plsc.subcore_barrier()

    pltpu.sync_copy(src_hbm.at[s], sidx_v)
    pltpu.sync_copy(dst_hbm.at[s], didx_v)

    def edge_chunk(j, carry):
        pltpu.async_copy(x2_hbm.at[c].at[sidx_v.at[j]], rows_v, sem).wait()
        pltpu.sync_copy(rows_v, sums_sh.at[didx_v.at[j]], add=True)

        @pl.when(c == 0)
        def _cnt():
            pltpu.sync_copy(ones_v, cnt_sh.at[didx_v.at[j]], add=True)

        return carry

    lax.fori_loop(0, RPW, edge_chunk, 0)
    plsc.subcore_barrier()

    for k in range(RPT // CH):
        r0 = s * RPT + k * CH
        pltpu.sync_copy(sums_sh.at[pl.ds(r0, CH)], sums_out.at[c, pl.ds(r0, CH)])

    @pl.when(c == 0)
    def _cnt_out():
        for k in range(RPT // CH):
            r0 = s * RPT + k * CH
            pltpu.sync_copy(cnt_sh.at[pl.ds(r0, CH)], cnt_out.at[pl.ds(r0, CH)])


def _sc_scatter(x2, src_p, dst_p):
    return pl.kernel(
        _sc_body,
        out_type=(
            jax.ShapeDtypeStruct((NC, TBL, HF), jnp.float32),
            jax.ShapeDtypeStruct((TBL, 16), jnp.float32),
        ),
        mesh=plsc.VectorSubcoreMesh(
            core_axis_name="c", subcore_axis_name="s",
            num_cores=NC, num_subcores=NS),
        compiler_params=pltpu.CompilerParams(use_tc_tiling_on_sc=False),
        scratch_types=[
            pltpu.VMEM((RPW, CH), jnp.int32),
            pltpu.VMEM((RPW, CH), jnp.int32),
            pltpu.VMEM((CH, HF), jnp.float32),
            pltpu.VMEM((CH, 16), jnp.float32),
            pltpu.VMEM((CH, 16), jnp.float32),
            pltpu.VMEM_SHARED((TBL, HF), jnp.float32),
            pltpu.VMEM_SHARED((TBL, 16), jnp.float32),
            pltpu.SemaphoreType.DMA,
        ],
    )(x2, src_p, dst_p)


def _tc_body(blo_ref, bhi_ref, x_ref, sp_ref, cp_ref, bcol_ref,
             wl_ref, bl_ref, wr_ref, w2_ref, b2_ref, out_ref, gacc):
    i = pl.program_id(0)

    @pl.when(i == 0)
    def _init():
        gacc[...] = jnp.full((NG, F), -jnp.inf, jnp.float32)

    sums = jnp.concatenate([sp_ref[0], sp_ref[1]], axis=1)
    cnt = cp_ref[:, 0:1]
    mean = sums / jnp.maximum(cnt, 1.0)
    h = (lax.dot_general(mean, wl_ref[...], (((1,), (1,)), ((), ())),
                         preferred_element_type=jnp.float32)
         + bl_ref[...]
         + lax.dot_general(x_ref[...], wr_ref[...], (((1,), (1,)), ((), ())),
                           preferred_element_type=jnp.float32))
    h = jnp.maximum(h, 0.0)
    rowid = i * 128 + lax.broadcasted_iota(jnp.int32, (128, 1), 0)
    h = jnp.where(rowid < N, h, -jnp.inf)

    bcol = bcol_ref[...]

    def upd(g, carry):
        contrib = jnp.max(jnp.where(bcol == g, h, -jnp.inf), axis=0,
                          keepdims=True)
        gacc[pl.ds(g, 1), :] = jnp.maximum(gacc[pl.ds(g, 1), :], contrib)
        return carry

    lax.fori_loop(blo_ref[i], bhi_ref[i] + 1, upd, 0)

    @pl.when(i == NBLK - 1)
    def _final():
        gv = gacc[...]
        gv = jnp.where(jnp.isfinite(gv), gv, 0.0)
        logits = lax.dot_general(gv, w2_ref[...], (((1,), (1,)), ((), ())),
                                 preferred_element_type=jnp.float32) + b2_ref[...]
        m = jnp.max(logits, axis=-1, keepdims=True)
        lse = jnp.log(jnp.sum(jnp.exp(logits - m), axis=-1, keepdims=True)) + m
        out_ref[...] = logits - lse


def _tc_dense(blo, bhi, xp, sums_part, cnt_tbl, bcol, W_l, b_l2, W_r, W2, b22):
    grid_spec = pltpu.PrefetchScalarGridSpec(
        num_scalar_prefetch=2,
        grid=(NBLK,),
        in_specs=[
            pl.BlockSpec((128, F), lambda i, *_: (i, 0)),
            pl.BlockSpec((NC, 128, HF), lambda i, *_: (0, i, 0)),
            pl.BlockSpec((128, 16), lambda i, *_: (i, 0)),
            pl.BlockSpec((128, 1), lambda i, *_: (i, 0)),
            pl.BlockSpec((F, F), lambda i, *_: (0, 0)),
            pl.BlockSpec((1, F), lambda i, *_: (0, 0)),
            pl.BlockSpec((F, F), lambda i, *_: (0, 0)),
            pl.BlockSpec((NCLS, F), lambda i, *_: (0, 0)),
            pl.BlockSpec((1, NCLS), lambda i, *_: (0, 0)),
        ],
        out_specs=pl.BlockSpec((NG, NCLS), lambda i, *_: (0, 0)),
        scratch_shapes=[pltpu.VMEM((NG, F), jnp.float32)],
    )
    return pl.pallas_call(
        _tc_body,
        grid_spec=grid_spec,
        out_shape=jax.ShapeDtypeStruct((NG, NCLS), jnp.float32),
        compiler_params=pltpu.CompilerParams(
            dimension_semantics=("arbitrary",)),
    )(blo, bhi, xp, sums_part, cnt_tbl, bcol, W_l, b_l2, W_r, W2, b22)


def kernel(x, edge_index, batch, W_l, b_l, W_r, W2, b2):
    src = edge_index[0]
    dst = edge_index[1]
    x2 = jnp.stack([x[:, :HF], x[:, HF:]])
    src_p = jnp.concatenate(
        [src, jnp.zeros((EPAD,), jnp.int32)]).reshape(NS, RPW, CH)
    dst_p = jnp.concatenate(
        [dst, jnp.full((EPAD,), DST_PAD, jnp.int32)]).reshape(NS, RPW, CH)

    sums_part, cnt_tbl = _sc_scatter(x2, src_p, dst_p)

    xp = jnp.concatenate([x, jnp.zeros((NPAD - N, F), jnp.float32)])
    batch_pad = jnp.concatenate(
        [batch, jnp.full((NPAD - N,), NG - 1, jnp.int32)])
    b2d = batch_pad.reshape(NBLK, 128)
    blo = b2d[:, 0]
    bhi = b2d[:, -1]
    bcol = batch_pad.reshape(NPAD, 1)

    return _tc_dense(blo, bhi, xp, sums_part, cnt_tbl, bcol,
                     W_l, b_l.reshape(1, F), W_r, W2, b2.reshape(1, NCLS))
